# baseline (speedup 1.0000x reference)
import functools
import math

import jax
import jax.numpy as jnp
from jax import lax
from jax.experimental import pallas as pl
from jax.experimental.pallas import tpu as pltpu
from jax.experimental.pallas import tpu_sc as plsc

_N = 20000
_C = 81
_CF = 80
_K = 200
_DETS = 100
_IMG_W = 1333.0
_IMG_H = 800.0
_SCORE_T = 0.05
_NMS_T = 0.5
_CLIP = math.log(1000.0 / 16.0)

_NPAD = 20480
_NW = 32
_PW = 512
_A_BLK = 2000


def _softmax_body(logit_ref, out_ref):
    x = logit_ref[...]
    m = jnp.max(x, axis=1, keepdims=True)
    e = jnp.exp(x - m)
    s = jnp.sum(e, axis=1, keepdims=True)
    p = e / s
    fg = p[:, 1:]
    masked = jnp.where(fg > _SCORE_T, fg, -1.0)
    pad = jnp.full((_CF, _NPAD - _N), -2.0, jnp.float32)
    out_ref[...] = jnp.concatenate([masked.T, pad], axis=1)


def _masked_scores(class_logits):
    return pl.pallas_call(
        _softmax_body,
        out_shape=jax.ShapeDtypeStruct((_CF, _NPAD), jnp.float32),
    )(class_logits)


_B05 = 1028443341
_B1 = 1065353216
_BN1 = -1082130432
_SLOTS = 1024
_TRASH = 1016
_NCH = _NPAD // 128


def _dest_body(sc_ref, dest_ref):
    kb = lax.bitcast_convert_type(sc_ref[...], jnp.int32)
    c05 = jnp.sum((kb > _B05).astype(jnp.int32), axis=1, keepdims=True)
    lo0 = jnp.full((_CF, 1), _B05, jnp.int32)
    hi0 = jnp.full((_CF, 1), _B1, jnp.int32)

    def bbody(t, carry):
        lo, hi = carry
        mid = (lo + hi) >> 1
        cnt = jnp.sum((kb > mid).astype(jnp.int32), axis=1, keepdims=True)
        small = cnt < _K
        return (jnp.where(small, lo, mid), jnp.where(small, mid, hi))

    lo, hi = lax.fori_loop(0, 26, bbody, (lo0, hi0))
    thr_bits = jnp.where(c05 >= _K, hi, jnp.int32(_BN1))

    r2 = lax.broadcasted_iota(jnp.int32, (128, 128), 0)
    c2 = lax.broadcasted_iota(jnp.int32, (128, 128), 1)
    ut = (r2 < c2).astype(jnp.float32)
    coff = (lax.broadcasted_iota(jnp.int32, (_CF, 1), 0) * _SLOTS).astype(jnp.float32)

    def cbody(i, carry):
        gc, tc = carry
        kchunk = lax.bitcast_convert_type(
            sc_ref[:, pl.ds(i * 128, 128)], jnp.int32)
        sg = kchunk > thr_bits
        st = kchunk == thr_bits
        sgf = sg.astype(jnp.float32)
        stf = st.astype(jnp.float32)
        grank = jnp.dot(sgf, ut, preferred_element_type=jnp.float32) + gc
        trank = jnp.dot(stf, ut, preferred_element_type=jnp.float32) + tc
        slot = jnp.where(
            sg, grank,
            jnp.where(st & (trank < float(_K)), 224.0 + trank, float(_TRASH)))
        dest_ref[:, pl.ds(i * 128, 128)] = (slot + coff).astype(jnp.int32)
        gc = gc + jnp.sum(sgf, axis=1, keepdims=True)
        tc = tc + jnp.sum(stf, axis=1, keepdims=True)
        return (gc, tc)

    z = jnp.zeros((_CF, 1), jnp.float32)
    lax.fori_loop(0, _NCH, cbody, (z, z))


def _dest(masked_p):
    return pl.pallas_call(
        _dest_body,
        out_shape=jax.ShapeDtypeStruct((_CF, _NPAD), jnp.int32),
    )(masked_p)


_GRP = 8


def _scatter_body(sc_hbm, dest_hbm, iota_hbm, cso_hbm, cio_hbm,
                  row_v, dest_v, iota_v, fill_v, ifill_v, sem):
    ci_ax = lax.axis_index("c")
    si_ax = lax.axis_index("s")
    w = si_ax * 2 + ci_ax
    pltpu.sync_copy(iota_hbm, iota_v)
    for j in range(_SLOTS // 16):
        fill_v[pl.ds(j * 16, 16)] = jnp.full((16,), -2.0)
        ifill_v[pl.ds(j * 16, 16)] = jnp.full((16,), 0, jnp.int32)
    nk = jnp.where(w < 16, 3, 2)
    base_c = jnp.where(w < 16, w * 3, 48 + (w - 16) * 2)
    for k in range(3):
        @pl.when(k < nk)
        def _():
            c = base_c + k
            pltpu.sync_copy(sc_hbm.at[c], row_v)
            pltpu.sync_copy(dest_hbm.at[c], dest_v)
            pltpu.sync_copy(fill_v, cso_hbm.at[pl.ds(c * _SLOTS, _SLOTS)])
            pltpu.sync_copy(ifill_v, cio_hbm.at[pl.ds(c * _SLOTS, _SLOTS)])

            def grp(g, carry):
                cps = []
                for jj in range(_GRP):
                    j = g * _GRP + jj
                    cp = pltpu.make_async_copy(
                        row_v.at[j], cso_hbm.at[dest_v.at[j]], sem)
                    cp.start()
                    cps.append(cp)
                    cp = pltpu.make_async_copy(
                        iota_v.at[j], cio_hbm.at[dest_v.at[j]], sem)
                    cp.start()
                    cps.append(cp)
                for cp in cps:
                    cp.wait()
                return carry

            lax.fori_loop(0, _NCH // _GRP, grp, 0)


def _scatter(masked_3, dest_3, iota_3):
    mesh = plsc.VectorSubcoreMesh(core_axis_name="c", subcore_axis_name="s")
    fn = functools.partial(
        pl.kernel,
        mesh=mesh,
        out_type=[
            jax.ShapeDtypeStruct((_CF * _SLOTS,), jnp.float32),
            jax.ShapeDtypeStruct((_CF * _SLOTS,), jnp.int32),
        ],
        scratch_types=[
            pltpu.VMEM((_NCH, 128), jnp.float32),
            pltpu.VMEM((_NCH, 128), jnp.int32),
            pltpu.VMEM((_NCH, 128), jnp.int32),
            pltpu.VMEM((_SLOTS,), jnp.float32),
            pltpu.VMEM((_SLOTS,), jnp.int32),
            pltpu.SemaphoreType.DMA,
        ],
    )(_scatter_body)
    return fn(masked_3, dest_3, iota_3)


def _gather_body(reg_hbm, prop_hbm, ridx_hbm, pidx_hbm, oreg_hbm, oprop_hbm,
                 idxr_v, idxp_v, regrows_v, proprows_v, sem):
    c = lax.axis_index("c")
    s = lax.axis_index("s")
    w = s * 2 + c
    pltpu.sync_copy(ridx_hbm.at[w], idxr_v)
    pltpu.sync_copy(pidx_hbm.at[w], idxp_v)
    copies = []
    for ch in range(4):
        for j in range(_PW // 128):
            cp = pltpu.make_async_copy(
                reg_hbm.at[idxr_v.at[ch, j]],
                regrows_v.at[ch, pl.ds(j * 128, 128)], sem)
            cp.start()
            copies.append(cp)
            cp = pltpu.make_async_copy(
                prop_hbm.at[idxp_v.at[ch, j]],
                proprows_v.at[ch, pl.ds(j * 128, 128)], sem)
            cp.start()
            copies.append(cp)
    for cp in copies:
        cp.wait()
    pltpu.sync_copy(regrows_v, oreg_hbm.at[w])
    pltpu.sync_copy(proprows_v, oprop_hbm.at[w])


def _gather_candidates(reg_flat, props, ridx, pidx):
    mesh = plsc.VectorSubcoreMesh(core_axis_name="c", subcore_axis_name="s")
    fn = functools.partial(
        pl.kernel,
        mesh=mesh,
        out_type=[
            jax.ShapeDtypeStruct((_NW, 4, _PW), jnp.float32),
            jax.ShapeDtypeStruct((_NW, 4, _PW), jnp.float32),
        ],
        scratch_types=[
            pltpu.VMEM((4, _PW // 128, 128), jnp.int32),
            pltpu.VMEM((4, _PW // 128, 128), jnp.int32),
            pltpu.VMEM((4, _PW), jnp.float32),
            pltpu.VMEM((4, _PW), jnp.float32),
            pltpu.SemaphoreType.DMA,
        ],
    )(_gather_body)
    return fn(reg_flat, props, ridx, pidx)


def _nms_body(sc_ref, reg_ref, prop_ref, outs_ref, outb_ref, area_ref, keep_ref):
    px1 = prop_ref[0]
    py1 = prop_ref[1]
    px2 = prop_ref[2]
    py2 = prop_ref[3]
    widths = px2 - px1 + 1.0
    heights = py2 - py1 + 1.0
    ctr_x = px1 + 0.5 * widths
    ctr_y = py1 + 0.5 * heights
    dx = reg_ref[0] / 10.0
    dy = reg_ref[1] / 10.0
    dw = jnp.minimum(reg_ref[2] / 5.0, _CLIP)
    dh = jnp.minimum(reg_ref[3] / 5.0, _CLIP)
    pred_ctr_x = dx * widths + ctr_x
    pred_ctr_y = dy * heights + ctr_y
    pred_w = jnp.exp(dw) * widths
    pred_h = jnp.exp(dh) * heights
    x1 = jnp.clip(pred_ctr_x - 0.5 * pred_w, 0.0, _IMG_W - 1.0)
    y1 = jnp.clip(pred_ctr_y - 0.5 * pred_h, 0.0, _IMG_H - 1.0)
    x2 = jnp.clip(pred_ctr_x + 0.5 * pred_w - 1.0, 0.0, _IMG_W - 1.0)
    y2 = jnp.clip(pred_ctr_y + 0.5 * pred_h - 1.0, 0.0, _IMG_H - 1.0)
    outb_ref[0] = x1
    outb_ref[1] = y1
    outb_ref[2] = x2
    outb_ref[3] = y2
    area_ref[...] = (x2 - x1 + 1.0) * (y2 - y1 + 1.0)
    area = area_ref[...]
    sc = sc_ref[...]
    keep_ref[...] = jnp.where(sc > _SCORE_T, 1.0, 0.0)
    row = lax.broadcasted_iota(jnp.int32, (_K, _CF), 0)

    def body(i, carry):
        a1 = outb_ref[0, pl.ds(i, 1), :]
        b1 = outb_ref[1, pl.ds(i, 1), :]
        a2 = outb_ref[2, pl.ds(i, 1), :]
        b2 = outb_ref[3, pl.ds(i, 1), :]
        ai = area_ref[pl.ds(i, 1), :]
        ltx = jnp.maximum(x1, a1)
        lty = jnp.maximum(y1, b1)
        rbx = jnp.minimum(x2, a2)
        rby = jnp.minimum(y2, b2)
        w = jnp.maximum(rbx - ltx + 1.0, 0.0)
        h = jnp.maximum(rby - lty + 1.0, 0.0)
        inter = w * h
        iou = inter / (area + ai - inter)
        keep = keep_ref[...]
        earlier = (keep > 0.5) & (row < i)
        sup = jnp.any((iou > _NMS_T) & earlier, axis=0, keepdims=True)
        ki = keep_ref[pl.ds(i, 1), :]
        keep_ref[pl.ds(i, 1), :] = jnp.where(sup, 0.0, ki)
        return carry

    lax.fori_loop(1, _K, body, 0)
    outs_ref[...] = jnp.where(keep_ref[...] > 0.5, sc, -1.0)


def _nms(sc_t, creg, cprop):
    return pl.pallas_call(
        _nms_body,
        out_shape=[
            jax.ShapeDtypeStruct((_K, _CF), jnp.float32),
            jax.ShapeDtypeStruct((4, _K, _CF), jnp.float32),
        ],
        scratch_shapes=[
            pltpu.VMEM((_K, _CF), jnp.float32),
            pltpu.VMEM((_K, _CF), jnp.float32),
        ],
    )(sc_t, creg, cprop)


@jax.jit
def kernel(class_logits, box_regression, proposals):
    masked_p = _masked_scores(class_logits)
    dest = _dest(masked_p)
    iota_3 = jnp.arange(_NPAD, dtype=jnp.int32).reshape(_NCH, 128)
    cso, cio = _scatter(masked_p.reshape(_CF, _NCH, 128),
                        dest.reshape(_CF, _NCH, 128), iota_3)
    cs = cso.reshape(_CF, _SLOTS)[:, : 424]
    ci = cio.reshape(_CF, _SLOTS)[:, : 424]
    top_scores, perm = lax.top_k(cs, _K)
    top_idx = jnp.take_along_axis(ci, perm, axis=1)

    cls = jnp.arange(1, _C, dtype=jnp.int32)[:, None]
    rrows = top_idx * _C + cls
    pad_n = _NW * _PW - _CF * _K
    pad_p = (jnp.arange(pad_n, dtype=jnp.int32) * 37) % _N
    rflat = jnp.concatenate([rrows.reshape(-1), pad_p * _C])
    pflat = jnp.concatenate([top_idx.reshape(-1), pad_p])
    ch_off = jnp.arange(4, dtype=jnp.int32)[:, None]
    ridx = (rflat[None, :] * 4 + ch_off).reshape(4, _NW, _PW // 128, 128)
    ridx = ridx.transpose(1, 0, 2, 3)
    pidx = (pflat[None, :] * 4 + ch_off).reshape(4, _NW, _PW // 128, 128)
    pidx = pidx.transpose(1, 0, 2, 3)

    reg_1d = box_regression.reshape(_N * _C * 4)
    prop_1d = proposals.reshape(_N * 4)
    oreg, oprop = _gather_candidates(reg_1d, prop_1d, ridx, pidx)

    creg = oreg.transpose(1, 0, 2).reshape(4, _NW * _PW)[:, : _CF * _K]
    creg = creg.reshape(4, _CF, _K).transpose(0, 2, 1)
    cprop = oprop.transpose(1, 0, 2).reshape(4, _NW * _PW)[:, : _CF * _K]
    cprop = cprop.reshape(4, _CF, _K).transpose(0, 2, 1)
    sc_t = top_scores.T

    outs, outb = _nms(sc_t, creg, cprop)

    flat_scores = outs.T.reshape(-1)
    flat_boxes = outb.transpose(2, 1, 0).reshape(_CF * _K, 4)
    fs, fi = lax.top_k(flat_scores, _DETS)
    top_boxes = flat_boxes[fi]
    top_labels = (fi // _K + 1).astype(jnp.int32)
    return top_boxes, fs, top_labels


# speedup vs baseline: 17.6205x; 17.6205x over previous
import functools
import math

import jax
import jax.numpy as jnp
from jax import lax
from jax.experimental import pallas as pl
from jax.experimental.pallas import tpu as pltpu
from jax.experimental.pallas import tpu_sc as plsc

_N = 20000
_C = 81
_CF = 80
_K = 200
_DETS = 100
_IMG_W = 1333.0
_IMG_H = 800.0
_SCORE_T = 0.05
_NMS_T = 0.5
_CLIP = math.log(1000.0 / 16.0)

_NPAD = 20480
_NW = 32
_PW = 512
_A_BLK = 2000


def _softmax_body(logit_ref, out_ref):
    x = logit_ref[...]
    m = jnp.max(x, axis=1, keepdims=True)
    e = jnp.exp(x - m)
    s = jnp.sum(e, axis=1, keepdims=True)
    p = e / s
    fg = p[:, 1:]
    masked = jnp.where(fg > _SCORE_T, fg, -1.0)
    pad = jnp.full((_CF, _NPAD - _N), -2.0, jnp.float32)
    out_ref[...] = jnp.concatenate([masked.T, pad], axis=1)


def _masked_scores(class_logits):
    return pl.pallas_call(
        _softmax_body,
        out_shape=jax.ShapeDtypeStruct((_CF, _NPAD), jnp.float32),
    )(class_logits)


_B05 = 1028443341
_B1 = 1065353216
_BN1 = -1082130432
_SLOTS = 1024
_TRASH = 1016
_NCH = _NPAD // 128


def _dest_body(sc_ref, dest_ref):
    kb = lax.bitcast_convert_type(sc_ref[...], jnp.int32)
    c05 = jnp.sum((kb > _B05).astype(jnp.int32), axis=1, keepdims=True)
    lo0 = jnp.full((_CF, 1), _B05, jnp.int32)
    hi0 = jnp.full((_CF, 1), _B1, jnp.int32)

    def bbody(t, carry):
        lo, hi = carry
        mid = (lo + hi) >> 1
        cnt = jnp.sum((kb > mid).astype(jnp.int32), axis=1, keepdims=True)
        small = cnt < _K
        return (jnp.where(small, lo, mid), jnp.where(small, mid, hi))

    lo, hi = lax.fori_loop(0, 26, bbody, (lo0, hi0))
    thr_bits = jnp.where(c05 >= _K, hi, jnp.int32(_BN1))

    r2 = lax.broadcasted_iota(jnp.int32, (128, 128), 0)
    c2 = lax.broadcasted_iota(jnp.int32, (128, 128), 1)
    ut = (r2 < c2).astype(jnp.float32)

    def cbody(i, carry):
        gc, tc = carry
        kchunk = lax.bitcast_convert_type(
            sc_ref[:, pl.ds(i * 128, 128)], jnp.int32)
        sg = kchunk > thr_bits
        st = kchunk == thr_bits
        sgf = sg.astype(jnp.float32)
        stf = st.astype(jnp.float32)
        grank = jnp.dot(sgf, ut, preferred_element_type=jnp.float32) + gc
        trank = jnp.dot(stf, ut, preferred_element_type=jnp.float32) + tc
        slot = jnp.where(
            sg, grank,
            jnp.where(st & (trank < float(_K)), 224.0 + trank, float(_TRASH)))
        dest_ref[:, pl.ds(i * 128, 128)] = slot.astype(jnp.int32)
        gc = gc + jnp.sum(sgf, axis=1, keepdims=True)
        tc = tc + jnp.sum(stf, axis=1, keepdims=True)
        return (gc, tc)

    z = jnp.zeros((_CF, 1), jnp.float32)
    lax.fori_loop(0, _NCH, cbody, (z, z))


def _dest(masked_p):
    return pl.pallas_call(
        _dest_body,
        out_shape=jax.ShapeDtypeStruct((_CF, _NPAD), jnp.int32),
    )(masked_p)


_GRP = 8


def _scatter_body(sc_hbm, dest_hbm, iota_hbm, cso_hbm, cio_hbm,
                  row_v, dest_v, iota_v, fill_v, ifill_v, shs_v, shi_v, sem):
    ci_ax = lax.axis_index("c")
    si_ax = lax.axis_index("s")
    w = si_ax * 2 + ci_ax
    pltpu.sync_copy(iota_hbm, iota_v)
    for j in range(_SLOTS // 16):
        fill_v[pl.ds(j * 16, 16)] = jnp.full((16,), -2.0)
        ifill_v[pl.ds(j * 16, 16)] = jnp.full((16,), 0, jnp.int32)
    nk = jnp.where(w < 16, 3, 2)
    base_c = jnp.where(w < 16, w * 3, 48 + (w - 16) * 2)
    for k in range(3):
        @pl.when(k < nk)
        def _():
            c = base_c + k
            region = (si_ax * 3 + k) * _SLOTS
            pltpu.sync_copy(sc_hbm.at[c], row_v)
            pltpu.sync_copy(dest_hbm.at[c], dest_v)
            roff = jnp.full((16,), region, jnp.int32)

            def addoff(i, carry):
                base = i * 16
                dflat = dest_v.at[i // 8]
                sl = pl.ds((i % 8) * 16, 16)
                dflat[sl] = dflat[sl] + roff
                return carry

            lax.fori_loop(0, _NCH * 8, addoff, 0)
            pltpu.sync_copy(fill_v, shs_v.at[pl.ds(region, _SLOTS)])
            pltpu.sync_copy(ifill_v, shi_v.at[pl.ds(region, _SLOTS)])

            def grp(g, carry):
                cps = []
                for jj in range(_GRP):
                    j = g * _GRP + jj
                    cp = pltpu.make_async_copy(
                        row_v.at[j], shs_v.at[dest_v.at[j]], sem)
                    cp.start()
                    cps.append(cp)
                    cp = pltpu.make_async_copy(
                        iota_v.at[j], shi_v.at[dest_v.at[j]], sem)
                    cp.start()
                    cps.append(cp)
                for cp in cps:
                    cp.wait()
                return carry

            lax.fori_loop(0, _NCH // _GRP, grp, 0)
            pltpu.sync_copy(shs_v.at[pl.ds(region, _SLOTS)],
                            cso_hbm.at[pl.ds(c * _SLOTS, _SLOTS)])
            pltpu.sync_copy(shi_v.at[pl.ds(region, _SLOTS)],
                            cio_hbm.at[pl.ds(c * _SLOTS, _SLOTS)])


def _scatter(masked_3, dest_3, iota_3):
    mesh = plsc.VectorSubcoreMesh(core_axis_name="c", subcore_axis_name="s")
    fn = functools.partial(
        pl.kernel,
        mesh=mesh,
        out_type=[
            jax.ShapeDtypeStruct((_CF * _SLOTS,), jnp.float32),
            jax.ShapeDtypeStruct((_CF * _SLOTS,), jnp.int32),
        ],
        scratch_types=[
            pltpu.VMEM((_NCH, 128), jnp.float32),
            pltpu.VMEM((_NCH, 128), jnp.int32),
            pltpu.VMEM((_NCH, 128), jnp.int32),
            pltpu.VMEM((_SLOTS,), jnp.float32),
            pltpu.VMEM((_SLOTS,), jnp.int32),
            pltpu.VMEM_SHARED((16 * 3 * _SLOTS,), jnp.float32),
            pltpu.VMEM_SHARED((16 * 3 * _SLOTS,), jnp.int32),
            pltpu.SemaphoreType.DMA,
        ],
    )(_scatter_body)
    return fn(masked_3, dest_3, iota_3)


def _gather_body(reg_hbm, prop_hbm, ridx_hbm, pidx_hbm, oreg_hbm, oprop_hbm,
                 idxr_v, idxp_v, regrows_v, proprows_v, sem):
    c = lax.axis_index("c")
    s = lax.axis_index("s")
    w = s * 2 + c
    pltpu.sync_copy(ridx_hbm.at[w], idxr_v)
    pltpu.sync_copy(pidx_hbm.at[w], idxp_v)
    copies = []
    for ch in range(4):
        for j in range(_PW // 128):
            cp = pltpu.make_async_copy(
                reg_hbm.at[idxr_v.at[ch, j]],
                regrows_v.at[ch, pl.ds(j * 128, 128)], sem)
            cp.start()
            copies.append(cp)
            cp = pltpu.make_async_copy(
                prop_hbm.at[idxp_v.at[ch, j]],
                proprows_v.at[ch, pl.ds(j * 128, 128)], sem)
            cp.start()
            copies.append(cp)
    for cp in copies:
        cp.wait()
    pltpu.sync_copy(regrows_v, oreg_hbm.at[w])
    pltpu.sync_copy(proprows_v, oprop_hbm.at[w])


def _gather_candidates(reg_flat, props, ridx, pidx):
    mesh = plsc.VectorSubcoreMesh(core_axis_name="c", subcore_axis_name="s")
    fn = functools.partial(
        pl.kernel,
        mesh=mesh,
        out_type=[
            jax.ShapeDtypeStruct((_NW, 4, _PW), jnp.float32),
            jax.ShapeDtypeStruct((_NW, 4, _PW), jnp.float32),
        ],
        scratch_types=[
            pltpu.VMEM((4, _PW // 128, 128), jnp.int32),
            pltpu.VMEM((4, _PW // 128, 128), jnp.int32),
            pltpu.VMEM((4, _PW), jnp.float32),
            pltpu.VMEM((4, _PW), jnp.float32),
            pltpu.SemaphoreType.DMA,
        ],
    )(_gather_body)
    return fn(reg_flat, props, ridx, pidx)


def _nms_body(sc_ref, reg_ref, prop_ref, outs_ref, outb_ref, area_ref, keep_ref):
    px1 = prop_ref[0]
    py1 = prop_ref[1]
    px2 = prop_ref[2]
    py2 = prop_ref[3]
    widths = px2 - px1 + 1.0
    heights = py2 - py1 + 1.0
    ctr_x = px1 + 0.5 * widths
    ctr_y = py1 + 0.5 * heights
    dx = reg_ref[0] / 10.0
    dy = reg_ref[1] / 10.0
    dw = jnp.minimum(reg_ref[2] / 5.0, _CLIP)
    dh = jnp.minimum(reg_ref[3] / 5.0, _CLIP)
    pred_ctr_x = dx * widths + ctr_x
    pred_ctr_y = dy * heights + ctr_y
    pred_w = jnp.exp(dw) * widths
    pred_h = jnp.exp(dh) * heights
    x1 = jnp.clip(pred_ctr_x - 0.5 * pred_w, 0.0, _IMG_W - 1.0)
    y1 = jnp.clip(pred_ctr_y - 0.5 * pred_h, 0.0, _IMG_H - 1.0)
    x2 = jnp.clip(pred_ctr_x + 0.5 * pred_w - 1.0, 0.0, _IMG_W - 1.0)
    y2 = jnp.clip(pred_ctr_y + 0.5 * pred_h - 1.0, 0.0, _IMG_H - 1.0)
    outb_ref[0] = x1
    outb_ref[1] = y1
    outb_ref[2] = x2
    outb_ref[3] = y2
    area_ref[...] = (x2 - x1 + 1.0) * (y2 - y1 + 1.0)
    area = area_ref[...]
    sc = sc_ref[...]
    keep_ref[...] = jnp.where(sc > _SCORE_T, 1.0, 0.0)
    row = lax.broadcasted_iota(jnp.int32, (_K, _CF), 0)

    def body(i, carry):
        a1 = outb_ref[0, pl.ds(i, 1), :]
        b1 = outb_ref[1, pl.ds(i, 1), :]
        a2 = outb_ref[2, pl.ds(i, 1), :]
        b2 = outb_ref[3, pl.ds(i, 1), :]
        ai = area_ref[pl.ds(i, 1), :]
        ltx = jnp.maximum(x1, a1)
        lty = jnp.maximum(y1, b1)
        rbx = jnp.minimum(x2, a2)
        rby = jnp.minimum(y2, b2)
        w = jnp.maximum(rbx - ltx + 1.0, 0.0)
        h = jnp.maximum(rby - lty + 1.0, 0.0)
        inter = w * h
        iou = inter / (area + ai - inter)
        keep = keep_ref[...]
        earlier = (keep > 0.5) & (row < i)
        sup = jnp.any((iou > _NMS_T) & earlier, axis=0, keepdims=True)
        ki = keep_ref[pl.ds(i, 1), :]
        keep_ref[pl.ds(i, 1), :] = jnp.where(sup, 0.0, ki)
        return carry

    lax.fori_loop(1, _K, body, 0)
    outs_ref[...] = jnp.where(keep_ref[...] > 0.5, sc, -1.0)


def _nms(sc_t, creg, cprop):
    return pl.pallas_call(
        _nms_body,
        out_shape=[
            jax.ShapeDtypeStruct((_K, _CF), jnp.float32),
            jax.ShapeDtypeStruct((4, _K, _CF), jnp.float32),
        ],
        scratch_shapes=[
            pltpu.VMEM((_K, _CF), jnp.float32),
            pltpu.VMEM((_K, _CF), jnp.float32),
        ],
    )(sc_t, creg, cprop)


@jax.jit
def kernel(class_logits, box_regression, proposals):
    masked_p = _masked_scores(class_logits)
    dest = _dest(masked_p)
    iota_3 = jnp.arange(_NPAD, dtype=jnp.int32).reshape(_NCH, 128)
    cso, cio = _scatter(masked_p.reshape(_CF, _NCH, 128),
                        dest.reshape(_CF, _NCH, 128), iota_3)
    cs = cso.reshape(_CF, _SLOTS)[:, : 424]
    ci = cio.reshape(_CF, _SLOTS)[:, : 424]
    top_scores, perm = lax.top_k(cs, _K)
    top_idx = jnp.take_along_axis(ci, perm, axis=1)

    cls = jnp.arange(1, _C, dtype=jnp.int32)[:, None]
    rrows = top_idx * _C + cls
    pad_n = _NW * _PW - _CF * _K
    pad_p = (jnp.arange(pad_n, dtype=jnp.int32) * 37) % _N
    rflat = jnp.concatenate([rrows.reshape(-1), pad_p * _C])
    pflat = jnp.concatenate([top_idx.reshape(-1), pad_p])
    ch_off = jnp.arange(4, dtype=jnp.int32)[:, None]
    ridx = (rflat[None, :] * 4 + ch_off).reshape(4, _NW, _PW // 128, 128)
    ridx = ridx.transpose(1, 0, 2, 3)
    pidx = (pflat[None, :] * 4 + ch_off).reshape(4, _NW, _PW // 128, 128)
    pidx = pidx.transpose(1, 0, 2, 3)

    reg_1d = box_regression.reshape(_N * _C * 4)
    prop_1d = proposals.reshape(_N * 4)
    oreg, oprop = _gather_candidates(reg_1d, prop_1d, ridx, pidx)

    creg = oreg.transpose(1, 0, 2).reshape(4, _NW * _PW)[:, : _CF * _K]
    creg = creg.reshape(4, _CF, _K).transpose(0, 2, 1)
    cprop = oprop.transpose(1, 0, 2).reshape(4, _NW * _PW)[:, : _CF * _K]
    cprop = cprop.reshape(4, _CF, _K).transpose(0, 2, 1)
    sc_t = top_scores.T

    outs, outb = _nms(sc_t, creg, cprop)

    flat_scores = outs.T.reshape(-1)
    flat_boxes = outb.transpose(2, 1, 0).reshape(_CF * _K, 4)
    fs, fi = lax.top_k(flat_scores, _DETS)
    top_boxes = flat_boxes[fi]
    top_labels = (fi // _K + 1).astype(jnp.int32)
    return top_boxes, fs, top_labels


# speedup vs baseline: 19.3022x; 1.0954x over previous
import functools
import math

import jax
import jax.numpy as jnp
from jax import lax
from jax.experimental import pallas as pl
from jax.experimental.pallas import tpu as pltpu
from jax.experimental.pallas import tpu_sc as plsc

_N = 20000
_C = 81
_CF = 80
_K = 200
_DETS = 100
_IMG_W = 1333.0
_IMG_H = 800.0
_SCORE_T = 0.05
_NMS_T = 0.5
_CLIP = math.log(1000.0 / 16.0)

_NPAD = 20480
_NW = 32
_PW = 512
_A_BLK = 2000


_B05 = 1028443341
_B1 = 1065353216
_BN1 = -1082130432
_SLOTS = 1024
_TRASH = 1016
_NCH = _NPAD // 128


def _prep_body(logit_ref, sc3_ref, dest_ref, m_ref):
    x = logit_ref[...]
    m = jnp.max(x, axis=1, keepdims=True)
    e = jnp.exp(x - m)
    s = jnp.sum(e, axis=1, keepdims=True)
    p = e / s
    fg = p[:, 1:]
    masked = jnp.where(fg > _SCORE_T, fg, -1.0)
    pad = jnp.full((_CF, _NPAD - _N), -2.0, jnp.float32)
    m_ref[...] = jnp.concatenate([masked.T, pad], axis=1)

    kb = lax.bitcast_convert_type(m_ref[...], jnp.int32)
    c05 = jnp.sum((kb > _B05).astype(jnp.int32), axis=1, keepdims=True)
    lo0 = jnp.full((_CF, 1), _B05, jnp.int32)
    hi0 = jnp.full((_CF, 1), _B1, jnp.int32)

    def bbody(t, carry):
        lo, hi = carry
        mid = (lo + hi) >> 1
        cnt = jnp.sum((kb > mid).astype(jnp.int32), axis=1, keepdims=True)
        small = cnt < _K
        return (jnp.where(small, lo, mid), jnp.where(small, mid, hi))

    lo, hi = lax.fori_loop(0, 26, bbody, (lo0, hi0))
    thr_bits = jnp.where(c05 >= _K, hi, jnp.int32(_BN1))

    r2 = lax.broadcasted_iota(jnp.int32, (128, 128), 0)
    c2 = lax.broadcasted_iota(jnp.int32, (128, 128), 1)
    ut = (r2 < c2).astype(jnp.float32)

    ci80 = lax.broadcasted_iota(jnp.int32, (_CF, 1), 0)
    r_lo = (ci80 // 3 // 2) * 3 + ci80 % 3
    r_hi = ((16 + (ci80 - 48) // 2) // 2) * 3 + (ci80 - 48) % 2
    coff = (jnp.where(ci80 < 48, r_lo, r_hi) * _SLOTS).astype(jnp.float32)

    def cbody(i, carry):
        gc, tc = carry
        mch = m_ref[:, pl.ds(i * 128, 128)]
        kchunk = lax.bitcast_convert_type(mch, jnp.int32)
        sg = kchunk > thr_bits
        st = kchunk == thr_bits
        sgf = sg.astype(jnp.float32)
        stf = st.astype(jnp.float32)
        grank = jnp.dot(sgf, ut, preferred_element_type=jnp.float32) + gc
        trank = jnp.dot(stf, ut, preferred_element_type=jnp.float32) + tc
        slot = jnp.where(
            sg, grank,
            jnp.where(st & (trank < float(_K)), 224.0 + trank, float(_TRASH)))
        sc3_ref[:, :, pl.ds(i * 128, 128)] = mch.reshape(_CF, 1, 128)
        dest_ref[:, :, pl.ds(i * 128, 128)] = (slot + coff).astype(jnp.int32).reshape(_CF, 1, 128)
        gc = gc + jnp.sum(sgf, axis=1, keepdims=True)
        tc = tc + jnp.sum(stf, axis=1, keepdims=True)
        return (gc, tc)

    z = jnp.zeros((_CF, 1), jnp.float32)
    lax.fori_loop(0, _NCH, cbody, (z, z))


def _prep(class_logits):
    return pl.pallas_call(
        _prep_body,
        out_shape=[
            jax.ShapeDtypeStruct((_CF, 1, _NPAD), jnp.float32),
            jax.ShapeDtypeStruct((_CF, 1, _NPAD), jnp.int32),
        ],
        scratch_shapes=[pltpu.VMEM((_CF, _NPAD), jnp.float32)],
    )(class_logits)


def _scatter_body(sc_hbm, dest_hbm, iota_hbm, cso_hbm, cio_hbm,
                  row_v, dest_v, iota_v, fill_v, ifill_v, shs_v, shi_v, sem):
    ci_ax = lax.axis_index("c")
    si_ax = lax.axis_index("s")
    w = si_ax * 2 + ci_ax
    pltpu.sync_copy(iota_hbm, iota_v)
    for j in range(_SLOTS // 16):
        fill_v[pl.ds(j * 16, 16)] = jnp.full((16,), -2.0)
        ifill_v[pl.ds(j * 16, 16)] = jnp.full((16,), 0, jnp.int32)
    nk = jnp.where(w < 16, 3, 2)
    base_c = jnp.where(w < 16, w * 3, 48 + (w - 16) * 2)
    for k in range(3):
        @pl.when(k < nk)
        def _():
            c = base_c + k
            region = (si_ax * 3 + k) * _SLOTS
            pltpu.sync_copy(sc_hbm.at[c], row_v)
            pltpu.sync_copy(dest_hbm.at[c], dest_v)
            pltpu.sync_copy(fill_v, shs_v.at[pl.ds(region, _SLOTS)])
            pltpu.sync_copy(ifill_v, shi_v.at[pl.ds(region, _SLOTS)])
            cp1 = pltpu.make_async_copy(row_v.at[0], shs_v.at[dest_v.at[0]], sem)
            cp1.start()
            cp2 = pltpu.make_async_copy(iota_v.at[0], shi_v.at[dest_v.at[0]], sem)
            cp2.start()
            cp1.wait()
            cp2.wait()
            pltpu.sync_copy(shs_v.at[pl.ds(region, _SLOTS)],
                            cso_hbm.at[pl.ds(c * _SLOTS, _SLOTS)])
            pltpu.sync_copy(shi_v.at[pl.ds(region, _SLOTS)],
                            cio_hbm.at[pl.ds(c * _SLOTS, _SLOTS)])


def _scatter(masked_3, dest_3, iota_3):
    mesh = plsc.VectorSubcoreMesh(core_axis_name="c", subcore_axis_name="s")
    fn = functools.partial(
        pl.kernel,
        mesh=mesh,
        out_type=[
            jax.ShapeDtypeStruct((_CF * _SLOTS,), jnp.float32),
            jax.ShapeDtypeStruct((_CF * _SLOTS,), jnp.int32),
        ],
        scratch_types=[
            pltpu.VMEM((1, _NPAD), jnp.float32),
            pltpu.VMEM((1, _NPAD), jnp.int32),
            pltpu.VMEM((1, _NPAD), jnp.int32),
            pltpu.VMEM((_SLOTS,), jnp.float32),
            pltpu.VMEM((_SLOTS,), jnp.int32),
            pltpu.VMEM_SHARED((16 * 3 * _SLOTS,), jnp.float32),
            pltpu.VMEM_SHARED((16 * 3 * _SLOTS,), jnp.int32),
            pltpu.SemaphoreType.DMA,
        ],
    )(_scatter_body)
    return fn(masked_3, dest_3, iota_3)


def _gather_body(reg_hbm, prop_hbm, ridx_hbm, pidx_hbm, oreg_hbm, oprop_hbm,
                 idxr_v, idxp_v, regrows_v, proprows_v, sem):
    c = lax.axis_index("c")
    s = lax.axis_index("s")
    w = s * 2 + c
    pltpu.sync_copy(ridx_hbm.at[w], idxr_v)
    pltpu.sync_copy(pidx_hbm.at[w], idxp_v)
    copies = []
    for ch in range(4):
        for j in range(_PW // 128):
            cp = pltpu.make_async_copy(
                reg_hbm.at[idxr_v.at[ch, j]],
                regrows_v.at[ch, pl.ds(j * 128, 128)], sem)
            cp.start()
            copies.append(cp)
            cp = pltpu.make_async_copy(
                prop_hbm.at[idxp_v.at[ch, j]],
                proprows_v.at[ch, pl.ds(j * 128, 128)], sem)
            cp.start()
            copies.append(cp)
    for cp in copies:
        cp.wait()
    pltpu.sync_copy(regrows_v, oreg_hbm.at[w])
    pltpu.sync_copy(proprows_v, oprop_hbm.at[w])


def _gather_candidates(reg_flat, props, ridx, pidx):
    mesh = plsc.VectorSubcoreMesh(core_axis_name="c", subcore_axis_name="s")
    fn = functools.partial(
        pl.kernel,
        mesh=mesh,
        out_type=[
            jax.ShapeDtypeStruct((_NW, 4, _PW), jnp.float32),
            jax.ShapeDtypeStruct((_NW, 4, _PW), jnp.float32),
        ],
        scratch_types=[
            pltpu.VMEM((4, _PW // 128, 128), jnp.int32),
            pltpu.VMEM((4, _PW // 128, 128), jnp.int32),
            pltpu.VMEM((4, _PW), jnp.float32),
            pltpu.VMEM((4, _PW), jnp.float32),
            pltpu.SemaphoreType.DMA,
        ],
    )(_gather_body)
    return fn(reg_flat, props, ridx, pidx)


def _nms_body(sc_ref, reg_ref, prop_ref, outs_ref, outb_ref, area_ref, keep_ref):
    px1 = prop_ref[0]
    py1 = prop_ref[1]
    px2 = prop_ref[2]
    py2 = prop_ref[3]
    widths = px2 - px1 + 1.0
    heights = py2 - py1 + 1.0
    ctr_x = px1 + 0.5 * widths
    ctr_y = py1 + 0.5 * heights
    dx = reg_ref[0] / 10.0
    dy = reg_ref[1] / 10.0
    dw = jnp.minimum(reg_ref[2] / 5.0, _CLIP)
    dh = jnp.minimum(reg_ref[3] / 5.0, _CLIP)
    pred_ctr_x = dx * widths + ctr_x
    pred_ctr_y = dy * heights + ctr_y
    pred_w = jnp.exp(dw) * widths
    pred_h = jnp.exp(dh) * heights
    x1 = jnp.clip(pred_ctr_x - 0.5 * pred_w, 0.0, _IMG_W - 1.0)
    y1 = jnp.clip(pred_ctr_y - 0.5 * pred_h, 0.0, _IMG_H - 1.0)
    x2 = jnp.clip(pred_ctr_x + 0.5 * pred_w - 1.0, 0.0, _IMG_W - 1.0)
    y2 = jnp.clip(pred_ctr_y + 0.5 * pred_h - 1.0, 0.0, _IMG_H - 1.0)
    outb_ref[0] = x1
    outb_ref[1] = y1
    outb_ref[2] = x2
    outb_ref[3] = y2
    area_ref[...] = (x2 - x1 + 1.0) * (y2 - y1 + 1.0)
    area = area_ref[...]
    sc = sc_ref[...]
    keep_ref[...] = jnp.where(sc > _SCORE_T, 1.0, 0.0)
    row = lax.broadcasted_iota(jnp.int32, (_K, _CF), 0)

    def body(i, carry):
        a1 = outb_ref[0, pl.ds(i, 1), :]
        b1 = outb_ref[1, pl.ds(i, 1), :]
        a2 = outb_ref[2, pl.ds(i, 1), :]
        b2 = outb_ref[3, pl.ds(i, 1), :]
        ai = area_ref[pl.ds(i, 1), :]
        ltx = jnp.maximum(x1, a1)
        lty = jnp.maximum(y1, b1)
        rbx = jnp.minimum(x2, a2)
        rby = jnp.minimum(y2, b2)
        w = jnp.maximum(rbx - ltx + 1.0, 0.0)
        h = jnp.maximum(rby - lty + 1.0, 0.0)
        inter = w * h
        iou = inter / (area + ai - inter)
        keep = keep_ref[...]
        earlier = (keep > 0.5) & (row < i)
        sup = jnp.any((iou > _NMS_T) & earlier, axis=0, keepdims=True)
        ki = keep_ref[pl.ds(i, 1), :]
        keep_ref[pl.ds(i, 1), :] = jnp.where(sup, 0.0, ki)
        return carry

    lax.fori_loop(1, _K, body, 0)
    outs_ref[...] = jnp.where(keep_ref[...] > 0.5, sc, -1.0)


def _nms(sc_t, creg, cprop):
    return pl.pallas_call(
        _nms_body,
        out_shape=[
            jax.ShapeDtypeStruct((_K, _CF), jnp.float32),
            jax.ShapeDtypeStruct((4, _K, _CF), jnp.float32),
        ],
        scratch_shapes=[
            pltpu.VMEM((_K, _CF), jnp.float32),
            pltpu.VMEM((_K, _CF), jnp.float32),
        ],
    )(sc_t, creg, cprop)


@jax.jit
def kernel(class_logits, box_regression, proposals):
    sc3, dest3 = _prep(class_logits)
    iota_3 = jnp.arange(_NPAD, dtype=jnp.int32).reshape(1, _NPAD)
    cso, cio = _scatter(sc3, dest3, iota_3)
    cs = cso.reshape(_CF, _SLOTS)[:, : 424]
    ci = cio.reshape(_CF, _SLOTS)[:, : 424]
    top_scores, perm = lax.top_k(cs, _K)
    top_idx = jnp.take_along_axis(ci, perm, axis=1)

    cls = jnp.arange(1, _C, dtype=jnp.int32)[:, None]
    rrows = top_idx * _C + cls
    pad_n = _NW * _PW - _CF * _K
    pad_p = (jnp.arange(pad_n, dtype=jnp.int32) * 37) % _N
    rflat = jnp.concatenate([rrows.reshape(-1), pad_p * _C])
    pflat = jnp.concatenate([top_idx.reshape(-1), pad_p])
    ch_off = jnp.arange(4, dtype=jnp.int32)[:, None]
    ridx = (rflat[None, :] * 4 + ch_off).reshape(4, _NW, _PW // 128, 128)
    ridx = ridx.transpose(1, 0, 2, 3)
    pidx = (pflat[None, :] * 4 + ch_off).reshape(4, _NW, _PW // 128, 128)
    pidx = pidx.transpose(1, 0, 2, 3)

    reg_1d = box_regression.reshape(_N * _C * 4)
    prop_1d = proposals.reshape(_N * 4)
    oreg, oprop = _gather_candidates(reg_1d, prop_1d, ridx, pidx)

    creg = oreg.transpose(1, 0, 2).reshape(4, _NW * _PW)[:, : _CF * _K]
    creg = creg.reshape(4, _CF, _K).transpose(0, 2, 1)
    cprop = oprop.transpose(1, 0, 2).reshape(4, _NW * _PW)[:, : _CF * _K]
    cprop = cprop.reshape(4, _CF, _K).transpose(0, 2, 1)
    sc_t = top_scores.T

    outs, outb = _nms(sc_t, creg, cprop)

    flat_scores = outs.T.reshape(-1)
    flat_boxes = outb.transpose(2, 1, 0).reshape(_CF * _K, 4)
    fs, fi = lax.top_k(flat_scores, _DETS)
    top_boxes = flat_boxes[fi]
    top_labels = (fi // _K + 1).astype(jnp.int32)
    return top_boxes, fs, top_labels


# speedup vs baseline: 83.2159x; 4.3112x over previous
import functools
import math

import jax
import jax.numpy as jnp
from jax import lax
from jax.experimental import pallas as pl
from jax.experimental.pallas import tpu as pltpu
from jax.experimental.pallas import tpu_sc as plsc

_N = 20000
_C = 81
_CF = 80
_K = 200
_DETS = 100
_IMG_W = 1333.0
_IMG_H = 800.0
_SCORE_T = 0.05
_NMS_T = 0.5
_CLIP = math.log(1000.0 / 16.0)

_NPAD = 20480
_NW = 32
_PW = 512
_A_BLK = 2000


_B05 = 1028443341
_B1 = 1065353216
_BN1 = -1082130432
_SLOTS = 1024
_TRASH = 1016
_NCH = _NPAD // 128


def _prep_body(logit_ref, sc3_ref, dest_ref, m_ref):
    x = logit_ref[...]
    m = jnp.max(x, axis=1, keepdims=True)
    e = jnp.exp(x - m)
    s = jnp.sum(e, axis=1, keepdims=True)
    p = e / s
    fg = p[:, 1:]
    masked = jnp.where(fg > _SCORE_T, fg, -1.0)
    pad = jnp.full((_CF, _NPAD - _N), -2.0, jnp.float32)
    m_ref[...] = jnp.concatenate([masked.T, pad], axis=1)

    kb = lax.bitcast_convert_type(m_ref[...], jnp.int32)
    c05 = jnp.sum((kb > _B05).astype(jnp.int32), axis=1, keepdims=True)
    lo0 = jnp.full((_CF, 1), _B05, jnp.int32)
    hi0 = jnp.full((_CF, 1), _B1, jnp.int32)

    def bbody(t, carry):
        lo, hi = carry
        mid = (lo + hi) >> 1
        cnt = jnp.sum((kb > mid).astype(jnp.int32), axis=1, keepdims=True)
        small = cnt < _K
        return (jnp.where(small, lo, mid), jnp.where(small, mid, hi))

    lo, hi = lax.fori_loop(0, 26, bbody, (lo0, hi0))
    thr_bits = jnp.where(c05 >= _K, hi, jnp.int32(_BN1))

    r2 = lax.broadcasted_iota(jnp.int32, (128, 128), 0)
    c2 = lax.broadcasted_iota(jnp.int32, (128, 128), 1)
    ut = (r2 < c2).astype(jnp.float32)
    lane = lax.broadcasted_iota(jnp.int32, (_CF, 128), 1).astype(jnp.float32)

    ci80 = lax.broadcasted_iota(jnp.int32, (_CF, 1), 0)
    r_lo = (ci80 // 3 // 2) * 3 + ci80 % 3
    r_hi = ((16 + (ci80 - 48) // 2) // 2) * 3 + (ci80 - 48) % 2
    coff = (jnp.where(ci80 < 48, r_lo, r_hi) * _SLOTS).astype(jnp.float32)

    def cbody(i, carry):
        gc, tc = carry
        mch = m_ref[:, pl.ds(i * 128, 128)]
        kchunk = lax.bitcast_convert_type(mch, jnp.int32)
        sg = kchunk > thr_bits
        st = kchunk == thr_bits
        sgf = sg.astype(jnp.float32)
        stf = st.astype(jnp.float32)
        grank = jnp.dot(sgf, ut, preferred_element_type=jnp.float32) + gc
        trank = jnp.dot(stf, ut, preferred_element_type=jnp.float32) + tc
        trash = (448.0 + (i % 4).astype(jnp.float32) * 128.0) + lane
        slot = jnp.where(
            sg, grank,
            jnp.where(st & (trank < float(_K)), 224.0 + trank, trash))
        sc3_ref[:, :, pl.ds(i * 128, 128)] = mch.reshape(_CF, 1, 128)
        dest_ref[:, :, pl.ds(i * 128, 128)] = (slot + coff).astype(jnp.int32).reshape(_CF, 1, 128)
        gc = gc + jnp.sum(sgf, axis=1, keepdims=True)
        tc = tc + jnp.sum(stf, axis=1, keepdims=True)
        return (gc, tc)

    z = jnp.zeros((_CF, 1), jnp.float32)
    lax.fori_loop(0, _NCH, cbody, (z, z))


def _prep(class_logits):
    return pl.pallas_call(
        _prep_body,
        out_shape=[
            jax.ShapeDtypeStruct((_CF, 1, _NPAD), jnp.float32),
            jax.ShapeDtypeStruct((_CF, 1, _NPAD), jnp.int32),
        ],
        scratch_shapes=[pltpu.VMEM((_CF, _NPAD), jnp.float32)],
    )(class_logits)


def _scatter_body(sc_hbm, dest_hbm, iota_hbm, cso_hbm, cio_hbm,
                  row_v, dest_v, iota_v, fill_v, ifill_v, shs_v, shi_v, sem):
    ci_ax = lax.axis_index("c")
    si_ax = lax.axis_index("s")
    w = si_ax * 2 + ci_ax
    pltpu.sync_copy(iota_hbm, iota_v)
    for j in range(_SLOTS // 16):
        fill_v[pl.ds(j * 16, 16)] = jnp.full((16,), -2.0)
        ifill_v[pl.ds(j * 16, 16)] = jnp.full((16,), 0, jnp.int32)
    nk = jnp.where(w < 16, 3, 2)
    base_c = jnp.where(w < 16, w * 3, 48 + (w - 16) * 2)
    for k in range(3):
        @pl.when(k < nk)
        def _():
            c = base_c + k
            region = (si_ax * 3 + k) * _SLOTS
            pltpu.sync_copy(sc_hbm.at[c], row_v)
            pltpu.sync_copy(dest_hbm.at[c], dest_v)
            pltpu.sync_copy(fill_v, shs_v.at[pl.ds(region, _SLOTS)])
            pltpu.sync_copy(ifill_v, shi_v.at[pl.ds(region, _SLOTS)])
            cp1 = pltpu.make_async_copy(row_v.at[0], shs_v.at[dest_v.at[0]], sem)
            cp1.start()
            cp2 = pltpu.make_async_copy(iota_v.at[0], shi_v.at[dest_v.at[0]], sem)
            cp2.start()
            cp1.wait()
            cp2.wait()
            pltpu.sync_copy(shs_v.at[pl.ds(region, _SLOTS)],
                            cso_hbm.at[pl.ds(c * _SLOTS, _SLOTS)])
            pltpu.sync_copy(shi_v.at[pl.ds(region, _SLOTS)],
                            cio_hbm.at[pl.ds(c * _SLOTS, _SLOTS)])


def _scatter(masked_3, dest_3, iota_3):
    mesh = plsc.VectorSubcoreMesh(core_axis_name="c", subcore_axis_name="s")
    fn = functools.partial(
        pl.kernel,
        mesh=mesh,
        out_type=[
            jax.ShapeDtypeStruct((_CF * _SLOTS,), jnp.float32),
            jax.ShapeDtypeStruct((_CF * _SLOTS,), jnp.int32),
        ],
        scratch_types=[
            pltpu.VMEM((1, _NPAD), jnp.float32),
            pltpu.VMEM((1, _NPAD), jnp.int32),
            pltpu.VMEM((1, _NPAD), jnp.int32),
            pltpu.VMEM((_SLOTS,), jnp.float32),
            pltpu.VMEM((_SLOTS,), jnp.int32),
            pltpu.VMEM_SHARED((16 * 3 * _SLOTS,), jnp.float32),
            pltpu.VMEM_SHARED((16 * 3 * _SLOTS,), jnp.int32),
            pltpu.SemaphoreType.DMA,
        ],
    )(_scatter_body)
    return fn(masked_3, dest_3, iota_3)


def _gather_body(reg_hbm, prop_hbm, ridx_hbm, pidx_hbm, oreg_hbm, oprop_hbm,
                 idxr_v, idxp_v, regrows_v, proprows_v, sem):
    c = lax.axis_index("c")
    s = lax.axis_index("s")
    w = s * 2 + c
    pltpu.sync_copy(ridx_hbm.at[w], idxr_v)
    pltpu.sync_copy(pidx_hbm.at[w], idxp_v)
    copies = []
    for ch in range(4):
        for j in range(_PW // 128):
            cp = pltpu.make_async_copy(
                reg_hbm.at[idxr_v.at[ch, j]],
                regrows_v.at[ch, pl.ds(j * 128, 128)], sem)
            cp.start()
            copies.append(cp)
            cp = pltpu.make_async_copy(
                prop_hbm.at[idxp_v.at[ch, j]],
                proprows_v.at[ch, pl.ds(j * 128, 128)], sem)
            cp.start()
            copies.append(cp)
    for cp in copies:
        cp.wait()
    pltpu.sync_copy(regrows_v, oreg_hbm.at[w])
    pltpu.sync_copy(proprows_v, oprop_hbm.at[w])


def _gather_candidates(reg_flat, props, ridx, pidx):
    mesh = plsc.VectorSubcoreMesh(core_axis_name="c", subcore_axis_name="s")
    fn = functools.partial(
        pl.kernel,
        mesh=mesh,
        out_type=[
            jax.ShapeDtypeStruct((_NW, 4, _PW), jnp.float32),
            jax.ShapeDtypeStruct((_NW, 4, _PW), jnp.float32),
        ],
        scratch_types=[
            pltpu.VMEM((4, _PW // 128, 128), jnp.int32),
            pltpu.VMEM((4, _PW // 128, 128), jnp.int32),
            pltpu.VMEM((4, _PW), jnp.float32),
            pltpu.VMEM((4, _PW), jnp.float32),
            pltpu.SemaphoreType.DMA,
        ],
    )(_gather_body)
    return fn(reg_flat, props, ridx, pidx)


def _nms_body(sc_ref, reg_ref, prop_ref, outs_ref, outb_ref, area_ref, keep_ref):
    px1 = prop_ref[0]
    py1 = prop_ref[1]
    px2 = prop_ref[2]
    py2 = prop_ref[3]
    widths = px2 - px1 + 1.0
    heights = py2 - py1 + 1.0
    ctr_x = px1 + 0.5 * widths
    ctr_y = py1 + 0.5 * heights
    dx = reg_ref[0] / 10.0
    dy = reg_ref[1] / 10.0
    dw = jnp.minimum(reg_ref[2] / 5.0, _CLIP)
    dh = jnp.minimum(reg_ref[3] / 5.0, _CLIP)
    pred_ctr_x = dx * widths + ctr_x
    pred_ctr_y = dy * heights + ctr_y
    pred_w = jnp.exp(dw) * widths
    pred_h = jnp.exp(dh) * heights
    x1 = jnp.clip(pred_ctr_x - 0.5 * pred_w, 0.0, _IMG_W - 1.0)
    y1 = jnp.clip(pred_ctr_y - 0.5 * pred_h, 0.0, _IMG_H - 1.0)
    x2 = jnp.clip(pred_ctr_x + 0.5 * pred_w - 1.0, 0.0, _IMG_W - 1.0)
    y2 = jnp.clip(pred_ctr_y + 0.5 * pred_h - 1.0, 0.0, _IMG_H - 1.0)
    outb_ref[0] = x1
    outb_ref[1] = y1
    outb_ref[2] = x2
    outb_ref[3] = y2
    area_ref[...] = (x2 - x1 + 1.0) * (y2 - y1 + 1.0)
    area = area_ref[...]
    sc = sc_ref[...]
    keep_ref[...] = jnp.where(sc > _SCORE_T, 1.0, 0.0)
    row = lax.broadcasted_iota(jnp.int32, (_K, _CF), 0)

    def body(i, carry):
        a1 = outb_ref[0, pl.ds(i, 1), :]
        b1 = outb_ref[1, pl.ds(i, 1), :]
        a2 = outb_ref[2, pl.ds(i, 1), :]
        b2 = outb_ref[3, pl.ds(i, 1), :]
        ai = area_ref[pl.ds(i, 1), :]
        ltx = jnp.maximum(x1, a1)
        lty = jnp.maximum(y1, b1)
        rbx = jnp.minimum(x2, a2)
        rby = jnp.minimum(y2, b2)
        w = jnp.maximum(rbx - ltx + 1.0, 0.0)
        h = jnp.maximum(rby - lty + 1.0, 0.0)
        inter = w * h
        iou = inter / (area + ai - inter)
        keep = keep_ref[...]
        earlier = (keep > 0.5) & (row < i)
        sup = jnp.any((iou > _NMS_T) & earlier, axis=0, keepdims=True)
        ki = keep_ref[pl.ds(i, 1), :]
        keep_ref[pl.ds(i, 1), :] = jnp.where(sup, 0.0, ki)
        return carry

    lax.fori_loop(1, _K, body, 0)
    outs_ref[...] = jnp.where(keep_ref[...] > 0.5, sc, -1.0)


def _nms(sc_t, creg, cprop):
    return pl.pallas_call(
        _nms_body,
        out_shape=[
            jax.ShapeDtypeStruct((_K, _CF), jnp.float32),
            jax.ShapeDtypeStruct((4, _K, _CF), jnp.float32),
        ],
        scratch_shapes=[
            pltpu.VMEM((_K, _CF), jnp.float32),
            pltpu.VMEM((_K, _CF), jnp.float32),
        ],
    )(sc_t, creg, cprop)


@jax.jit
def kernel(class_logits, box_regression, proposals):
    sc3, dest3 = _prep(class_logits)
    iota_3 = jnp.arange(_NPAD, dtype=jnp.int32).reshape(1, _NPAD)
    cso, cio = _scatter(sc3, dest3, iota_3)
    cs = cso.reshape(_CF, _SLOTS)[:, : 424]
    ci = cio.reshape(_CF, _SLOTS)[:, : 424]
    top_scores, perm = lax.top_k(cs, _K)
    top_idx = jnp.take_along_axis(ci, perm, axis=1)

    cls = jnp.arange(1, _C, dtype=jnp.int32)[:, None]
    rrows = top_idx * _C + cls
    pad_n = _NW * _PW - _CF * _K
    pad_p = (jnp.arange(pad_n, dtype=jnp.int32) * 37) % _N
    rflat = jnp.concatenate([rrows.reshape(-1), pad_p * _C])
    pflat = jnp.concatenate([top_idx.reshape(-1), pad_p])
    ch_off = jnp.arange(4, dtype=jnp.int32)[:, None]
    ridx = (rflat[None, :] * 4 + ch_off).reshape(4, _NW, _PW // 128, 128)
    ridx = ridx.transpose(1, 0, 2, 3)
    pidx = (pflat[None, :] * 4 + ch_off).reshape(4, _NW, _PW // 128, 128)
    pidx = pidx.transpose(1, 0, 2, 3)

    reg_1d = box_regression.reshape(_N * _C * 4)
    prop_1d = proposals.reshape(_N * 4)
    oreg, oprop = _gather_candidates(reg_1d, prop_1d, ridx, pidx)

    creg = oreg.transpose(1, 0, 2).reshape(4, _NW * _PW)[:, : _CF * _K]
    creg = creg.reshape(4, _CF, _K).transpose(0, 2, 1)
    cprop = oprop.transpose(1, 0, 2).reshape(4, _NW * _PW)[:, : _CF * _K]
    cprop = cprop.reshape(4, _CF, _K).transpose(0, 2, 1)
    sc_t = top_scores.T

    outs, outb = _nms(sc_t, creg, cprop)

    flat_scores = outs.T.reshape(-1)
    flat_boxes = outb.transpose(2, 1, 0).reshape(_CF * _K, 4)
    fs, fi = lax.top_k(flat_scores, _DETS)
    top_boxes = flat_boxes[fi]
    top_labels = (fi // _K + 1).astype(jnp.int32)
    return top_boxes, fs, top_labels


# speedup vs baseline: 83.3483x; 1.0016x over previous
import functools
import math

import jax
import jax.numpy as jnp
from jax import lax
from jax.experimental import pallas as pl
from jax.experimental.pallas import tpu as pltpu
from jax.experimental.pallas import tpu_sc as plsc

_N = 20000
_C = 81
_CF = 80
_K = 200
_DETS = 100
_IMG_W = 1333.0
_IMG_H = 800.0
_SCORE_T = 0.05
_NMS_T = 0.5
_CLIP = math.log(1000.0 / 16.0)

_NPAD = 20480
_NW = 32
_PW = 512
_A_BLK = 2000


_B05 = 1028443341
_B1 = 1065353216
_BN1 = -1082130432
_SLOTS = 1024
_TRASH = 1016
_NCH = _NPAD // 128


def _prep_body(logit_ref, sc3_ref, dest_ref, m_ref):
    x = logit_ref[...]
    m = jnp.max(x, axis=1, keepdims=True)
    e = jnp.exp(x - m)
    s = jnp.sum(e, axis=1, keepdims=True)
    p = e / s
    fg = p[:, 1:]
    masked = jnp.where(fg > _SCORE_T, fg, -1.0)
    pad = jnp.full((_CF, _NPAD - _N), -2.0, jnp.float32)
    m_ref[...] = jnp.concatenate([masked.T, pad], axis=1)

    kb = lax.bitcast_convert_type(m_ref[...], jnp.int32)
    c05 = jnp.sum((kb > _B05).astype(jnp.int32), axis=1, keepdims=True)
    lo0 = jnp.full((_CF, 1), _B05, jnp.int32)
    hi0 = jnp.full((_CF, 1), _B1, jnp.int32)

    def bbody(t, carry):
        lo, hi = carry
        mid = (lo + hi) >> 1
        cnt = jnp.sum((kb > mid).astype(jnp.int32), axis=1, keepdims=True)
        small = cnt < _K
        return (jnp.where(small, lo, mid), jnp.where(small, mid, hi))

    lo, hi = lax.fori_loop(0, 26, bbody, (lo0, hi0))
    thr_bits = jnp.where(c05 >= _K, hi, jnp.int32(_BN1))

    r2 = lax.broadcasted_iota(jnp.int32, (128, 128), 0)
    c2 = lax.broadcasted_iota(jnp.int32, (128, 128), 1)
    ut = (r2 < c2).astype(jnp.float32)
    lane = lax.broadcasted_iota(jnp.int32, (_CF, 128), 1).astype(jnp.float32)

    ci80 = lax.broadcasted_iota(jnp.int32, (_CF, 1), 0)
    r_lo = (ci80 // 3 // 2) * 3 + ci80 % 3
    r_hi = ((16 + (ci80 - 48) // 2) // 2) * 3 + (ci80 - 48) % 2
    coff = (jnp.where(ci80 < 48, r_lo, r_hi) * _SLOTS).astype(jnp.float32)

    def cbody(i, carry):
        gc, tc = carry
        mch = m_ref[:, pl.ds(i * 128, 128)]
        kchunk = lax.bitcast_convert_type(mch, jnp.int32)
        sg = kchunk > thr_bits
        st = kchunk == thr_bits
        sgf = sg.astype(jnp.float32)
        stf = st.astype(jnp.float32)
        grank = jnp.dot(sgf, ut, preferred_element_type=jnp.float32) + gc
        trank = jnp.dot(stf, ut, preferred_element_type=jnp.float32) + tc
        trash = (448.0 + (i % 4).astype(jnp.float32) * 128.0) + lane
        slot = jnp.where(
            sg, grank,
            jnp.where(st & (trank < float(_K)), 224.0 + trank, trash))
        sc3_ref[:, pl.ds(i, 1), :] = mch.reshape(_CF, 1, 128)
        dest_ref[:, pl.ds(i, 1), :] = (slot + coff).astype(jnp.int32).reshape(_CF, 1, 128)
        gc = gc + jnp.sum(sgf, axis=1, keepdims=True)
        tc = tc + jnp.sum(stf, axis=1, keepdims=True)
        return (gc, tc)

    z = jnp.zeros((_CF, 1), jnp.float32)
    lax.fori_loop(0, _NCH, cbody, (z, z))


def _prep(class_logits):
    return pl.pallas_call(
        _prep_body,
        out_shape=[
            jax.ShapeDtypeStruct((_CF, _NCH, 128), jnp.float32),
            jax.ShapeDtypeStruct((_CF, _NCH, 128), jnp.int32),
        ],
        scratch_shapes=[pltpu.VMEM((_CF, _NPAD), jnp.float32)],
    )(class_logits)


def _scatter_body(sc_hbm, dest_hbm, iota_hbm, cso_hbm, cio_hbm,
                  row_v, dest_v, iota_v, fill_v, ifill_v, shs_v, shi_v, sem):
    ci_ax = lax.axis_index("c")
    si_ax = lax.axis_index("s")
    w = si_ax * 2 + ci_ax
    pltpu.sync_copy(iota_hbm, iota_v)
    for j in range(_SLOTS // 16):
        fill_v[pl.ds(j * 16, 16)] = jnp.full((16,), -2.0)
        ifill_v[pl.ds(j * 16, 16)] = jnp.full((16,), 0, jnp.int32)
    nk = jnp.where(w < 16, 3, 2)
    base_c = jnp.where(w < 16, w * 3, 48 + (w - 16) * 2)
    for k in range(3):
        @pl.when(k < nk)
        def _():
            c = base_c + k
            region = (si_ax * 3 + k) * _SLOTS
            pltpu.sync_copy(sc_hbm.at[pl.ds(c * _NPAD, _NPAD)], row_v)
            pltpu.sync_copy(dest_hbm.at[pl.ds(c * _NPAD, _NPAD)], dest_v)
            pltpu.sync_copy(fill_v, shs_v.at[pl.ds(region, _SLOTS)])
            pltpu.sync_copy(ifill_v, shi_v.at[pl.ds(region, _SLOTS)])
            cp1 = pltpu.make_async_copy(row_v, shs_v.at[dest_v], sem)
            cp1.start()
            cp2 = pltpu.make_async_copy(iota_v, shi_v.at[dest_v], sem)
            cp2.start()
            cp1.wait()
            cp2.wait()
            pltpu.sync_copy(shs_v.at[pl.ds(region, _SLOTS)],
                            cso_hbm.at[pl.ds(c * _SLOTS, _SLOTS)])
            pltpu.sync_copy(shi_v.at[pl.ds(region, _SLOTS)],
                            cio_hbm.at[pl.ds(c * _SLOTS, _SLOTS)])


def _scatter(masked_3, dest_3, iota_3):
    mesh = plsc.VectorSubcoreMesh(core_axis_name="c", subcore_axis_name="s")
    fn = functools.partial(
        pl.kernel,
        mesh=mesh,
        out_type=[
            jax.ShapeDtypeStruct((_CF * _SLOTS,), jnp.float32),
            jax.ShapeDtypeStruct((_CF * _SLOTS,), jnp.int32),
        ],
        scratch_types=[
            pltpu.VMEM((_NPAD,), jnp.float32),
            pltpu.VMEM((_NPAD,), jnp.int32),
            pltpu.VMEM((_NPAD,), jnp.int32),
            pltpu.VMEM((_SLOTS,), jnp.float32),
            pltpu.VMEM((_SLOTS,), jnp.int32),
            pltpu.VMEM_SHARED((16 * 3 * _SLOTS,), jnp.float32),
            pltpu.VMEM_SHARED((16 * 3 * _SLOTS,), jnp.int32),
            pltpu.SemaphoreType.DMA,
        ],
    )(_scatter_body)
    return fn(masked_3, dest_3, iota_3)


def _gather_body(reg_hbm, prop_hbm, ridx_hbm, pidx_hbm, oreg_hbm, oprop_hbm,
                 idxr_v, idxp_v, regrows_v, proprows_v, sem):
    c = lax.axis_index("c")
    s = lax.axis_index("s")
    w = s * 2 + c
    pltpu.sync_copy(ridx_hbm.at[w], idxr_v)
    pltpu.sync_copy(pidx_hbm.at[w], idxp_v)
    copies = []
    for ch in range(4):
        for j in range(_PW // 128):
            cp = pltpu.make_async_copy(
                reg_hbm.at[idxr_v.at[ch, j]],
                regrows_v.at[ch, pl.ds(j * 128, 128)], sem)
            cp.start()
            copies.append(cp)
            cp = pltpu.make_async_copy(
                prop_hbm.at[idxp_v.at[ch, j]],
                proprows_v.at[ch, pl.ds(j * 128, 128)], sem)
            cp.start()
            copies.append(cp)
    for cp in copies:
        cp.wait()
    pltpu.sync_copy(regrows_v, oreg_hbm.at[w])
    pltpu.sync_copy(proprows_v, oprop_hbm.at[w])


def _gather_candidates(reg_flat, props, ridx, pidx):
    mesh = plsc.VectorSubcoreMesh(core_axis_name="c", subcore_axis_name="s")
    fn = functools.partial(
        pl.kernel,
        mesh=mesh,
        out_type=[
            jax.ShapeDtypeStruct((_NW, 4, _PW), jnp.float32),
            jax.ShapeDtypeStruct((_NW, 4, _PW), jnp.float32),
        ],
        scratch_types=[
            pltpu.VMEM((4, _PW // 128, 128), jnp.int32),
            pltpu.VMEM((4, _PW // 128, 128), jnp.int32),
            pltpu.VMEM((4, _PW), jnp.float32),
            pltpu.VMEM((4, _PW), jnp.float32),
            pltpu.SemaphoreType.DMA,
        ],
    )(_gather_body)
    return fn(reg_flat, props, ridx, pidx)


def _nms_body(sc_ref, reg_ref, prop_ref, outs_ref, outb_ref, area_ref, keep_ref):
    px1 = prop_ref[0]
    py1 = prop_ref[1]
    px2 = prop_ref[2]
    py2 = prop_ref[3]
    widths = px2 - px1 + 1.0
    heights = py2 - py1 + 1.0
    ctr_x = px1 + 0.5 * widths
    ctr_y = py1 + 0.5 * heights
    dx = reg_ref[0] / 10.0
    dy = reg_ref[1] / 10.0
    dw = jnp.minimum(reg_ref[2] / 5.0, _CLIP)
    dh = jnp.minimum(reg_ref[3] / 5.0, _CLIP)
    pred_ctr_x = dx * widths + ctr_x
    pred_ctr_y = dy * heights + ctr_y
    pred_w = jnp.exp(dw) * widths
    pred_h = jnp.exp(dh) * heights
    x1 = jnp.clip(pred_ctr_x - 0.5 * pred_w, 0.0, _IMG_W - 1.0)
    y1 = jnp.clip(pred_ctr_y - 0.5 * pred_h, 0.0, _IMG_H - 1.0)
    x2 = jnp.clip(pred_ctr_x + 0.5 * pred_w - 1.0, 0.0, _IMG_W - 1.0)
    y2 = jnp.clip(pred_ctr_y + 0.5 * pred_h - 1.0, 0.0, _IMG_H - 1.0)
    outb_ref[0] = x1
    outb_ref[1] = y1
    outb_ref[2] = x2
    outb_ref[3] = y2
    area_ref[...] = (x2 - x1 + 1.0) * (y2 - y1 + 1.0)
    area = area_ref[...]
    sc = sc_ref[...]
    keep_ref[...] = jnp.where(sc > _SCORE_T, 1.0, 0.0)
    row = lax.broadcasted_iota(jnp.int32, (_K, _CF), 0)

    def body(i, carry):
        a1 = outb_ref[0, pl.ds(i, 1), :]
        b1 = outb_ref[1, pl.ds(i, 1), :]
        a2 = outb_ref[2, pl.ds(i, 1), :]
        b2 = outb_ref[3, pl.ds(i, 1), :]
        ai = area_ref[pl.ds(i, 1), :]
        ltx = jnp.maximum(x1, a1)
        lty = jnp.maximum(y1, b1)
        rbx = jnp.minimum(x2, a2)
        rby = jnp.minimum(y2, b2)
        w = jnp.maximum(rbx - ltx + 1.0, 0.0)
        h = jnp.maximum(rby - lty + 1.0, 0.0)
        inter = w * h
        iou = inter / (area + ai - inter)
        keep = keep_ref[...]
        earlier = (keep > 0.5) & (row < i)
        sup = jnp.any((iou > _NMS_T) & earlier, axis=0, keepdims=True)
        ki = keep_ref[pl.ds(i, 1), :]
        keep_ref[pl.ds(i, 1), :] = jnp.where(sup, 0.0, ki)
        return carry

    lax.fori_loop(1, _K, body, 0)
    outs_ref[...] = jnp.where(keep_ref[...] > 0.5, sc, -1.0)


def _nms(sc_t, creg, cprop):
    return pl.pallas_call(
        _nms_body,
        out_shape=[
            jax.ShapeDtypeStruct((_K, _CF), jnp.float32),
            jax.ShapeDtypeStruct((4, _K, _CF), jnp.float32),
        ],
        scratch_shapes=[
            pltpu.VMEM((_K, _CF), jnp.float32),
            pltpu.VMEM((_K, _CF), jnp.float32),
        ],
    )(sc_t, creg, cprop)


@jax.jit
def kernel(class_logits, box_regression, proposals):
    sc3, dest3 = _prep(class_logits)
    iota_1 = jnp.arange(_NPAD, dtype=jnp.int32)
    cso, cio = _scatter(sc3.reshape(-1), dest3.reshape(-1), iota_1)
    cs = cso.reshape(_CF, _SLOTS)[:, : 424]
    ci = cio.reshape(_CF, _SLOTS)[:, : 424]
    top_scores, perm = lax.top_k(cs, _K)
    top_idx = jnp.take_along_axis(ci, perm, axis=1)

    cls = jnp.arange(1, _C, dtype=jnp.int32)[:, None]
    rrows = top_idx * _C + cls
    pad_n = _NW * _PW - _CF * _K
    pad_p = (jnp.arange(pad_n, dtype=jnp.int32) * 37) % _N
    rflat = jnp.concatenate([rrows.reshape(-1), pad_p * _C])
    pflat = jnp.concatenate([top_idx.reshape(-1), pad_p])
    ch_off = jnp.arange(4, dtype=jnp.int32)[:, None]
    ridx = (rflat[None, :] * 4 + ch_off).reshape(4, _NW, _PW // 128, 128)
    ridx = ridx.transpose(1, 0, 2, 3)
    pidx = (pflat[None, :] * 4 + ch_off).reshape(4, _NW, _PW // 128, 128)
    pidx = pidx.transpose(1, 0, 2, 3)

    reg_1d = box_regression.reshape(_N * _C * 4)
    prop_1d = proposals.reshape(_N * 4)
    oreg, oprop = _gather_candidates(reg_1d, prop_1d, ridx, pidx)

    creg = oreg.transpose(1, 0, 2).reshape(4, _NW * _PW)[:, : _CF * _K]
    creg = creg.reshape(4, _CF, _K).transpose(0, 2, 1)
    cprop = oprop.transpose(1, 0, 2).reshape(4, _NW * _PW)[:, : _CF * _K]
    cprop = cprop.reshape(4, _CF, _K).transpose(0, 2, 1)
    sc_t = top_scores.T

    outs, outb = _nms(sc_t, creg, cprop)

    flat_scores = outs.T.reshape(-1)
    flat_boxes = outb.transpose(2, 1, 0).reshape(_CF * _K, 4)
    fs, fi = lax.top_k(flat_scores, _DETS)
    top_boxes = flat_boxes[fi]
    top_labels = (fi // _K + 1).astype(jnp.int32)
    return top_boxes, fs, top_labels


# speedup vs baseline: 83.4125x; 1.0008x over previous
import functools
import math

import jax
import jax.numpy as jnp
from jax import lax
from jax.experimental import pallas as pl
from jax.experimental.pallas import tpu as pltpu
from jax.experimental.pallas import tpu_sc as plsc

_N = 20000
_C = 81
_CF = 80
_K = 200
_DETS = 100
_IMG_W = 1333.0
_IMG_H = 800.0
_SCORE_T = 0.05
_NMS_T = 0.5
_CLIP = math.log(1000.0 / 16.0)

_NPAD = 20480
_NW = 32
_PW = 512
_A_BLK = 2000


_B05 = 1028443341
_B1 = 1065353216
_BN1 = -1082130432
_SLOTS = 1024
_TRASH = 1016
_NCH = _NPAD // 128


def _prep_body(logit_ref, sc3_ref, dest_ref, m_ref):
    x = logit_ref[...]
    m = jnp.max(x, axis=1, keepdims=True)
    e = jnp.exp(x - m)
    s = jnp.sum(e, axis=1, keepdims=True)
    p = e / s
    fg = p[:, 1:]
    masked = jnp.where(fg > _SCORE_T, fg, -1.0)
    pad = jnp.full((_CF, _NPAD - _N), -2.0, jnp.float32)
    m_ref[...] = jnp.concatenate([masked.T, pad], axis=1)

    kb = lax.bitcast_convert_type(m_ref[...], jnp.int32)
    c05 = jnp.sum((kb > _B05).astype(jnp.int32), axis=1, keepdims=True)
    lo0 = jnp.full((_CF, 1), _B05, jnp.int32)
    hi0 = jnp.full((_CF, 1), _B1, jnp.int32)

    def bbody(t, carry):
        lo, hi = carry
        mid = (lo + hi) >> 1
        cnt = jnp.sum((kb > mid).astype(jnp.int32), axis=1, keepdims=True)
        small = cnt < _K
        return (jnp.where(small, lo, mid), jnp.where(small, mid, hi))

    lo, hi = lax.fori_loop(0, 26, bbody, (lo0, hi0))
    thr_bits = jnp.where(c05 >= _K, hi, jnp.int32(_BN1))

    r2 = lax.broadcasted_iota(jnp.int32, (128, 128), 0)
    c2 = lax.broadcasted_iota(jnp.int32, (128, 128), 1)
    ut = (r2 < c2).astype(jnp.float32)
    lane = lax.broadcasted_iota(jnp.int32, (_CF, 128), 1).astype(jnp.float32)

    ci80 = lax.broadcasted_iota(jnp.int32, (_CF, 1), 0)
    r_lo = (ci80 // 3 // 2) * 3 + ci80 % 3
    r_hi = ((16 + (ci80 - 48) // 2) // 2) * 3 + (ci80 - 48) % 2
    coff = (jnp.where(ci80 < 48, r_lo, r_hi) * _SLOTS).astype(jnp.float32)

    def cbody(i, carry):
        gc, tc = carry
        mch = m_ref[:, pl.ds(i * 128, 128)]
        kchunk = lax.bitcast_convert_type(mch, jnp.int32)
        sg = kchunk > thr_bits
        st = kchunk == thr_bits
        sgf = sg.astype(jnp.float32)
        stf = st.astype(jnp.float32)
        grank = jnp.dot(sgf, ut, preferred_element_type=jnp.float32) + gc
        trank = jnp.dot(stf, ut, preferred_element_type=jnp.float32) + tc
        trash = (448.0 + (i % 4).astype(jnp.float32) * 128.0) + lane
        slot = jnp.where(
            sg, grank,
            jnp.where(st & (trank < float(_K)), 224.0 + trank, trash))
        sc3_ref[:, pl.ds(i, 1), :] = mch.reshape(_CF, 1, 128)
        dest_ref[:, pl.ds(i, 1), :] = (slot + coff).astype(jnp.int32).reshape(_CF, 1, 128)
        gc = gc + jnp.sum(sgf, axis=1, keepdims=True)
        tc = tc + jnp.sum(stf, axis=1, keepdims=True)
        return (gc, tc)

    z = jnp.zeros((_CF, 1), jnp.float32)
    lax.fori_loop(0, _NCH, cbody, (z, z))


def _prep(class_logits):
    return pl.pallas_call(
        _prep_body,
        out_shape=[
            jax.ShapeDtypeStruct((_CF, _NCH, 128), jnp.float32),
            jax.ShapeDtypeStruct((_CF, _NCH, 128), jnp.int32),
        ],
        scratch_shapes=[pltpu.VMEM((_CF, _NPAD), jnp.float32)],
    )(class_logits)


def _scatter_body(sc_hbm, dest_hbm, iota_hbm, cso_hbm, cio_hbm,
                  row_v, dest_v, iota_v, fill_v, ifill_v, shs_v, shi_v, sem):
    ci_ax = lax.axis_index("c")
    si_ax = lax.axis_index("s")
    w = si_ax * 2 + ci_ax
    pltpu.sync_copy(iota_hbm, iota_v)
    for j in range(_SLOTS // 16):
        fill_v[pl.ds(j * 16, 16)] = jnp.full((16,), -2.0)
        ifill_v[pl.ds(j * 16, 16)] = jnp.full((16,), 0, jnp.int32)
    nk = jnp.where(w < 16, 3, 2)
    base_c = jnp.where(w < 16, w * 3, 48 + (w - 16) * 2)
    for k in range(3):
        @pl.when(k < nk)
        def _():
            c = base_c + k
            region = (si_ax * 3 + k) * _SLOTS
            pltpu.sync_copy(sc_hbm.at[pl.ds(c * _NPAD, _NPAD)], row_v)
            pltpu.sync_copy(dest_hbm.at[pl.ds(c * _NPAD, _NPAD)], dest_v)
            pltpu.sync_copy(fill_v, shs_v.at[pl.ds(region, _SLOTS)])
            pltpu.sync_copy(ifill_v, shi_v.at[pl.ds(region, _SLOTS)])
            cp1 = pltpu.make_async_copy(row_v, shs_v.at[dest_v], sem)
            cp1.start()
            cp2 = pltpu.make_async_copy(iota_v, shi_v.at[dest_v], sem)
            cp2.start()
            cp1.wait()
            cp2.wait()
            pltpu.sync_copy(shs_v.at[pl.ds(region, _SLOTS)],
                            cso_hbm.at[pl.ds(c * _SLOTS, _SLOTS)])
            pltpu.sync_copy(shi_v.at[pl.ds(region, _SLOTS)],
                            cio_hbm.at[pl.ds(c * _SLOTS, _SLOTS)])


def _scatter(masked_3, dest_3, iota_3):
    mesh = plsc.VectorSubcoreMesh(core_axis_name="c", subcore_axis_name="s")
    fn = functools.partial(
        pl.kernel,
        mesh=mesh,
        out_type=[
            jax.ShapeDtypeStruct((_CF * _SLOTS,), jnp.float32),
            jax.ShapeDtypeStruct((_CF * _SLOTS,), jnp.int32),
        ],
        scratch_types=[
            pltpu.VMEM((_NPAD,), jnp.float32),
            pltpu.VMEM((_NPAD,), jnp.int32),
            pltpu.VMEM((_NPAD,), jnp.int32),
            pltpu.VMEM((_SLOTS,), jnp.float32),
            pltpu.VMEM((_SLOTS,), jnp.int32),
            pltpu.VMEM_SHARED((16 * 3 * _SLOTS,), jnp.float32),
            pltpu.VMEM_SHARED((16 * 3 * _SLOTS,), jnp.int32),
            pltpu.SemaphoreType.DMA,
        ],
    )(_scatter_body)
    return fn(masked_3, dest_3, iota_3)


def _gather_body(reg_hbm, prop_hbm, ridx_hbm, pidx_hbm, oreg_hbm, oprop_hbm,
                 idxr_v, idxp_v, regrows_v, proprows_v, sem):
    c = lax.axis_index("c")
    s = lax.axis_index("s")
    w = s * 2 + c
    pltpu.sync_copy(ridx_hbm.at[w], idxr_v)
    pltpu.sync_copy(pidx_hbm.at[w], idxp_v)
    copies = []
    for ch in range(4):
        for j in range(_PW // 128):
            cp = pltpu.make_async_copy(
                reg_hbm.at[idxr_v.at[ch, j]],
                regrows_v.at[ch, pl.ds(j * 128, 128)], sem)
            cp.start()
            copies.append(cp)
            cp = pltpu.make_async_copy(
                prop_hbm.at[idxp_v.at[ch, j]],
                proprows_v.at[ch, pl.ds(j * 128, 128)], sem)
            cp.start()
            copies.append(cp)
    for cp in copies:
        cp.wait()
    pltpu.sync_copy(regrows_v, oreg_hbm.at[w])
    pltpu.sync_copy(proprows_v, oprop_hbm.at[w])


def _gather_candidates(reg_flat, props, ridx, pidx):
    mesh = plsc.VectorSubcoreMesh(core_axis_name="c", subcore_axis_name="s")
    fn = functools.partial(
        pl.kernel,
        mesh=mesh,
        out_type=[
            jax.ShapeDtypeStruct((_NW, 4, _PW), jnp.float32),
            jax.ShapeDtypeStruct((_NW, 4, _PW), jnp.float32),
        ],
        scratch_types=[
            pltpu.VMEM((4, _PW // 128, 128), jnp.int32),
            pltpu.VMEM((4, _PW // 128, 128), jnp.int32),
            pltpu.VMEM((4, _PW), jnp.float32),
            pltpu.VMEM((4, _PW), jnp.float32),
            pltpu.SemaphoreType.DMA,
        ],
    )(_gather_body)
    return fn(reg_flat, props, ridx, pidx)


def _nms_body(sc_ref, reg_ref, prop_ref, outs_ref, outb_ref, area_ref, keep_ref):
    px1 = prop_ref[0]
    py1 = prop_ref[1]
    px2 = prop_ref[2]
    py2 = prop_ref[3]
    widths = px2 - px1 + 1.0
    heights = py2 - py1 + 1.0
    ctr_x = px1 + 0.5 * widths
    ctr_y = py1 + 0.5 * heights
    dx = reg_ref[0] / 10.0
    dy = reg_ref[1] / 10.0
    dw = jnp.minimum(reg_ref[2] / 5.0, _CLIP)
    dh = jnp.minimum(reg_ref[3] / 5.0, _CLIP)
    pred_ctr_x = dx * widths + ctr_x
    pred_ctr_y = dy * heights + ctr_y
    pred_w = jnp.exp(dw) * widths
    pred_h = jnp.exp(dh) * heights
    x1 = jnp.clip(pred_ctr_x - 0.5 * pred_w, 0.0, _IMG_W - 1.0)
    y1 = jnp.clip(pred_ctr_y - 0.5 * pred_h, 0.0, _IMG_H - 1.0)
    x2 = jnp.clip(pred_ctr_x + 0.5 * pred_w - 1.0, 0.0, _IMG_W - 1.0)
    y2 = jnp.clip(pred_ctr_y + 0.5 * pred_h - 1.0, 0.0, _IMG_H - 1.0)
    outb_ref[0] = x1
    outb_ref[1] = y1
    outb_ref[2] = x2
    outb_ref[3] = y2
    area_ref[...] = (x2 - x1 + 1.0) * (y2 - y1 + 1.0)
    area = area_ref[...]
    sc = sc_ref[...]
    keep_ref[...] = jnp.where(sc > _SCORE_T, 1.0, 0.0)
    row = lax.broadcasted_iota(jnp.int32, (_K, _CF), 0)

    def body(i, carry):
        a1 = outb_ref[0, pl.ds(i, 1), :]
        b1 = outb_ref[1, pl.ds(i, 1), :]
        a2 = outb_ref[2, pl.ds(i, 1), :]
        b2 = outb_ref[3, pl.ds(i, 1), :]
        ai = area_ref[pl.ds(i, 1), :]
        ltx = jnp.maximum(x1, a1)
        lty = jnp.maximum(y1, b1)
        rbx = jnp.minimum(x2, a2)
        rby = jnp.minimum(y2, b2)
        w = jnp.maximum(rbx - ltx + 1.0, 0.0)
        h = jnp.maximum(rby - lty + 1.0, 0.0)
        inter = w * h
        iou = inter / (area + ai - inter)
        keep = keep_ref[...]
        earlier = (keep > 0.5) & (row < i)
        sup = jnp.any((iou > _NMS_T) & earlier, axis=0, keepdims=True)
        ki = keep_ref[pl.ds(i, 1), :]
        keep_ref[pl.ds(i, 1), :] = jnp.where(sup, 0.0, ki)
        return carry

    lax.fori_loop(1, _K, body, 0)
    outs_ref[...] = jnp.where(keep_ref[...] > 0.5, sc, -1.0)


def _nms(sc_t, creg, cprop):
    return pl.pallas_call(
        _nms_body,
        out_shape=[
            jax.ShapeDtypeStruct((_K, _CF), jnp.float32),
            jax.ShapeDtypeStruct((4, _K, _CF), jnp.float32),
        ],
        scratch_shapes=[
            pltpu.VMEM((_K, _CF), jnp.float32),
            pltpu.VMEM((_K, _CF), jnp.float32),
        ],
    )(sc_t, creg, cprop)


@jax.jit
def kernel(class_logits, box_regression, proposals):
    sc3, dest3 = _prep(class_logits)
    iota_1 = jnp.arange(_NPAD, dtype=jnp.int32)
    cso, cio = _scatter(sc3.reshape(-1), dest3.reshape(-1), iota_1)
    cs = cso.reshape(_CF, _SLOTS)[:, : 424]
    ci = cio.reshape(_CF, _SLOTS)[:, : 424]
    top_scores, perm = lax.top_k(cs, _K)
    top_idx = jnp.take_along_axis(ci, perm, axis=1)

    cls = jnp.arange(1, _C, dtype=jnp.int32)[:, None]
    rrows = top_idx * _C + cls
    pad_n = _NW * _PW - _CF * _K
    pad_p = (jnp.arange(pad_n, dtype=jnp.int32) * 37) % _N
    rflat = jnp.concatenate([rrows.reshape(-1), pad_p * _C])
    pflat = jnp.concatenate([top_idx.reshape(-1), pad_p])
    ch_off = jnp.arange(4, dtype=jnp.int32)[:, None]
    ridx = (rflat[None, :] * 4 + ch_off).reshape(4, _NW, _PW // 128, 128)
    ridx = ridx.transpose(1, 0, 2, 3)
    pidx = (pflat[None, :] * 4 + ch_off).reshape(4, _NW, _PW // 128, 128)
    pidx = pidx.transpose(1, 0, 2, 3)

    reg_1d = jax.lax.optimization_barrier(box_regression * 1.0).reshape(_N * _C * 4)
    prop_1d = proposals.reshape(_N * 4)
    oreg, oprop = _gather_candidates(reg_1d, prop_1d, ridx, pidx)

    creg = oreg.transpose(1, 0, 2).reshape(4, _NW * _PW)[:, : _CF * _K]
    creg = creg.reshape(4, _CF, _K).transpose(0, 2, 1)
    cprop = oprop.transpose(1, 0, 2).reshape(4, _NW * _PW)[:, : _CF * _K]
    cprop = cprop.reshape(4, _CF, _K).transpose(0, 2, 1)
    sc_t = top_scores.T

    outs, outb = _nms(sc_t, creg, cprop)

    flat_scores = outs.T.reshape(-1)
    flat_boxes = outb.transpose(2, 1, 0).reshape(_CF * _K, 4)
    fs, fi = lax.top_k(flat_scores, _DETS)
    top_boxes = flat_boxes[fi]
    top_labels = (fi // _K + 1).astype(jnp.int32)
    return top_boxes, fs, top_labels
